# Initial kernel scaffold; baseline (speedup 1.0000x reference)
import jax
import jax.numpy as jnp
from jax.experimental import pallas as pl


def kernel(nfeatures, efeatures, edge_index, params):
    raise NotImplementedError("write your pallas kernel here")


# speedup vs baseline: 2.6395x; 2.6395x over previous
import functools

import jax
import jax.numpy as jnp
from jax import lax
from jax.experimental import pallas as pl
from jax.experimental.pallas import tpu as pltpu
from jax.experimental.pallas import tpu_sc as plsc

_N = 100000
_E = 1600000
_F = 16

_NC = 2
_NS = 16
_NW = _NC * _NS
_CH = 128
_NCHW = 392
_EPW = _NCHW * _CH
_E_PAD = _EPW * _NW
_TOT_CH = _E // _CH
_N_ACC = 100480
_NPS = _N_ACC // _NS

_BRE = 8000
_BRN = 10000


def _leaky(x):
    return jnp.where(x >= 0, x, 0.01 * x)


def _real_chunks(wid):
    return jnp.clip(_TOT_CH - wid * _NCHW, 0, _NCHW)


def _gather_sc(pn, src2, dst2):
    mesh = plsc.VectorSubcoreMesh(core_axis_name="c", subcore_axis_name="s")

    @functools.partial(
        pl.kernel,
        out_type=(jax.ShapeDtypeStruct((_E, _F), jnp.float32),
                  jax.ShapeDtypeStruct((_E, _F), jnp.float32)),
        mesh=mesh,
        scratch_types=[
            pltpu.VMEM((_NCHW, _CH), jnp.int32),
            pltpu.VMEM((_NCHW, _CH), jnp.int32),
            pltpu.VMEM((_CH, _F), jnp.float32),
            pltpu.VMEM((_CH, _F), jnp.float32),
            pltpu.SemaphoreType.DMA,
            pltpu.SemaphoreType.DMA,
        ],
        compiler_params=pltpu.CompilerParams(use_tc_tiling_on_sc=False),
    )
    def k(pn_hbm, src_hbm, dst_hbm, s_out, d_out, sidx, didx, srows, drows,
          sem_s, sem_d):
        wid = lax.axis_index("s") * _NC + lax.axis_index("c")
        ch0 = wid * _NCHW
        nch = _real_chunks(wid)
        pltpu.sync_copy(src_hbm.at[pl.ds(ch0, _NCHW)], sidx)
        pltpu.sync_copy(dst_hbm.at[pl.ds(ch0, _NCHW)], didx)

        def body(g, _):
            row0 = (ch0 + g) * _CH
            cs = pltpu.async_copy(pn_hbm.at[sidx.at[g]], srows, sem_s)
            cd = pltpu.async_copy(pn_hbm.at[didx.at[g]], drows, sem_d)
            cs.wait()
            cd.wait()
            pltpu.sync_copy(srows, s_out.at[pl.ds(row0, _CH)])
            pltpu.sync_copy(drows, d_out.at[pl.ds(row0, _CH)])
            return 0

        lax.fori_loop(0, nch, body, 0)

    return k(pn, src2, dst2)


def _scatter_sc(pe, dst1, zeros):
    mesh = plsc.VectorSubcoreMesh(core_axis_name="c", subcore_axis_name="s")

    @functools.partial(
        pl.kernel,
        out_type=jax.ShapeDtypeStruct((_NC, _N_ACC, _F), jnp.float32),
        mesh=mesh,
        scratch_types=[
            pltpu.VMEM((_CH,), jnp.int32),
            pltpu.VMEM((_CH, _F), jnp.float32),
            pltpu.VMEM_SHARED((_N_ACC, _F), jnp.float32),
        ],
        compiler_params=pltpu.CompilerParams(use_tc_tiling_on_sc=False),
    )
    def k(pe_hbm, dst_hbm, z_hbm, out_hbm, didx, rows, acc):
        c = lax.axis_index("c")
        s = lax.axis_index("s")
        wid = s * _NC + c
        ch0 = wid * _NCHW
        nch = _real_chunks(wid)
        pltpu.sync_copy(z_hbm.at[pl.ds(s * _NPS, _NPS)],
                        acc.at[pl.ds(s * _NPS, _NPS)])
        plsc.subcore_barrier()

        def body(g, _):
            row0 = (ch0 + g) * _CH
            pltpu.sync_copy(dst_hbm.at[pl.ds(row0, _CH)], didx)
            pltpu.sync_copy(pe_hbm.at[pl.ds(row0, _CH)], rows)
            pltpu.sync_copy(rows, acc.at[didx], add=True)
            return 0

        lax.fori_loop(0, nch, body, 0)
        plsc.subcore_barrier()
        pltpu.sync_copy(acc.at[pl.ds(s * _NPS, _NPS)],
                        out_hbm.at[c].at[pl.ds(s * _NPS, _NPS)])

    return k(pe, dst1, zeros)


def _mlp_tc(p, xs, acc3=None, res=None, br=_BRE, w1_row_splits=None):
    R = xs[0].shape[0]
    grid = R // br
    assert grid * br == R
    w1, b1 = p['inp']
    w2, b2 = p['hidden'][0]
    w3, b3 = p['out']
    fo = w3.shape[1]
    ln = p.get('ln')

    splits = []
    off = 0
    for x in xs:
        splits.append(w1[off:off + x.shape[1]])
        off += x.shape[1]
    w1_acc = w1[off:] if acc3 is not None else None

    nx = len(xs)
    has_acc = acc3 is not None
    has_res = res is not None
    has_ln = ln is not None

    def body(*refs):
        i = 0
        xr = refs[i:i + nx]; i += nx
        if has_acc:
            a0 = refs[i]; a1 = refs[i + 1]; i += 2
        w1r = refs[i:i + nx]; i += nx
        if has_acc:
            w1ar = refs[i]; i += 1
        b1r, w2r, b2r, w3r, b3r = refs[i:i + 5]; i += 5
        if has_ln:
            gr, br_ = refs[i], refs[i + 1]; i += 2
        if has_res:
            rr = refs[i]; i += 1
        out = refs[i]

        f = b1r[...]
        for x, w in zip(xr, w1r):
            f = f + jnp.dot(x[...], w[...], preferred_element_type=jnp.float32)
        if has_acc:
            asum = a0[0] + a1[0]
            f = f + jnp.dot(asum, w1ar[...], preferred_element_type=jnp.float32)
        f = _leaky(f)
        f = _leaky(jnp.dot(f, w2r[...], preferred_element_type=jnp.float32) + b2r[...])
        f = jnp.dot(f, w3r[...], preferred_element_type=jnp.float32) + b3r[...]
        if has_ln:
            mu = jnp.mean(f, axis=1, keepdims=True)
            var = jnp.mean((f - mu) ** 2, axis=1, keepdims=True)
            f = (f - mu) * lax.rsqrt(var + 1e-5) * gr[...] + br_[...]
        if has_res:
            f = f + rr[...]
        out[...] = f

    full = lambda a: pl.BlockSpec(a.shape, lambda i: tuple(0 for _ in a.shape))
    rowblk = lambda a: pl.BlockSpec((br, a.shape[1]), lambda i: (i, 0))

    operands = []
    in_specs = []
    for x in xs:
        operands.append(x); in_specs.append(rowblk(x))
    if has_acc:
        operands += [acc3, acc3]
        in_specs += [pl.BlockSpec((1, br, _F), lambda i: (0, i, 0)),
                     pl.BlockSpec((1, br, _F), lambda i: (1, i, 0))]
    for w in splits:
        operands.append(w); in_specs.append(full(w))
    if has_acc:
        operands.append(w1_acc); in_specs.append(full(w1_acc))
    for a in (b1.reshape(1, -1), w2, b2.reshape(1, -1), w3, b3.reshape(1, -1)):
        operands.append(a); in_specs.append(full(a))
    if has_ln:
        for a in (ln[0].reshape(1, -1), ln[1].reshape(1, -1)):
            operands.append(a); in_specs.append(full(a))
    if has_res:
        operands.append(res); in_specs.append(rowblk(res))

    return pl.pallas_call(
        body,
        grid=(grid,),
        in_specs=in_specs,
        out_specs=pl.BlockSpec((br, fo), lambda i: (i, 0)),
        out_shape=jax.ShapeDtypeStruct((R, fo), jnp.float32),
        compiler_params=pltpu.CompilerParams(
            dimension_semantics=("arbitrary",)),
    )(*operands)


def kernel(nfeatures, efeatures, edge_index, params):
    pad = _E_PAD - _E
    src2 = jnp.pad(edge_index[0], (0, pad)).reshape(_NW * _NCHW, _CH)
    dst1 = jnp.pad(edge_index[1], (0, pad))
    dst2 = dst1.reshape(_NW * _NCHW, _CH)
    zeros = jnp.zeros((_N_ACC, _F), jnp.float32)

    pn = _mlp_tc(params['enc_n'], [nfeatures], br=_BRN)
    pe = _mlp_tc(params['enc_e'], [efeatures], br=_BRE)
    for i in range(2):
        s, d = _gather_sc(pn, src2, dst2)
        pe = _mlp_tc(params['proc_e'][i], [pe, s, d], res=pe, br=_BRE)
        parts = _scatter_sc(pe, dst1, zeros)
        pn = _mlp_tc(params['proc_n'][i], [pn], acc3=parts, res=pn, br=_BRN)
    return _mlp_tc(params['out'], [pn], br=_BRN)


# speedup vs baseline: 3.0046x; 1.1383x over previous
import functools

import jax
import jax.numpy as jnp
from jax import lax
from jax.experimental import pallas as pl
from jax.experimental.pallas import tpu as pltpu
from jax.experimental.pallas import tpu_sc as plsc

_N = 100000
_E = 1600000
_F = 16

_NC = 2
_NS = 16
_NW = _NC * _NS
_CH = 128
_NCHW = 392
_EPW = _NCHW * _CH
_E_PAD = _EPW * _NW
_TOT_CH = _E // _CH
_N_ACC = 100480
_NPS = _N_ACC // _NS

_BRE = 8000
_BRN = 10000


def _leaky(x):
    return jnp.where(x >= 0, x, 0.01 * x)


def _real_chunks(wid):
    return jnp.clip(_TOT_CH - wid * _NCHW, 0, _NCHW)


_GK = 4
_GR = _GK * _CH
_RS = 3


def _gather_sc(pn, src2, dst2):
    mesh = plsc.VectorSubcoreMesh(core_axis_name="c", subcore_axis_name="s")

    @functools.partial(
        pl.kernel,
        out_type=(jax.ShapeDtypeStruct((_E, _F), jnp.float32),
                  jax.ShapeDtypeStruct((_E, _F), jnp.float32)),
        mesh=mesh,
        scratch_types=[
            pltpu.VMEM((_RS, _GK, _CH), jnp.int32),
            pltpu.VMEM((_RS, _GK, _CH), jnp.int32),
            pltpu.VMEM((_RS, _GR, _F), jnp.float32),
            pltpu.VMEM((_RS, _GR, _F), jnp.float32),
            pltpu.SemaphoreType.DMA((_RS,)),
            pltpu.SemaphoreType.DMA((_RS,)),
            pltpu.SemaphoreType.DMA((_RS,)),
        ],
        compiler_params=pltpu.CompilerParams(use_tc_tiling_on_sc=False),
    )
    def k(pn_hbm, src_hbm, dst_hbm, s_out, d_out, sidx, didx, srows, drows,
          isem, gsem, wsem):
        wid = lax.axis_index("s") * _NC + lax.axis_index("c")
        ch0 = wid * _NCHW
        ng = _real_chunks(wid) // _GK

        def stage_idx(g, slot):
            pltpu.async_copy(src_hbm.at[pl.ds(ch0 + g * _GK, _GK)],
                             sidx.at[slot], isem.at[slot])
            pltpu.async_copy(dst_hbm.at[pl.ds(ch0 + g * _GK, _GK)],
                             didx.at[slot], isem.at[slot])

        def wait_idx(g, slot):
            pltpu.make_async_copy(src_hbm.at[pl.ds(ch0 + g * _GK, _GK)],
                                  sidx.at[slot], isem.at[slot]).wait()
            pltpu.make_async_copy(dst_hbm.at[pl.ds(ch0 + g * _GK, _GK)],
                                  didx.at[slot], isem.at[slot]).wait()

        def fire_gathers(slot):
            for j in range(_GK):
                pltpu.async_copy(pn_hbm.at[sidx.at[slot, j]],
                                 srows.at[slot, pl.ds(j * _CH, _CH)],
                                 gsem.at[slot])
                pltpu.async_copy(pn_hbm.at[didx.at[slot, j]],
                                 drows.at[slot, pl.ds(j * _CH, _CH)],
                                 gsem.at[slot])

        def wait_gathers(g, slot):
            row0 = (ch0 + g * _GK) * _CH
            pltpu.make_async_copy(s_out.at[pl.ds(row0, _GR)], srows.at[slot],
                                  gsem.at[slot]).wait()
            pltpu.make_async_copy(s_out.at[pl.ds(row0, _GR)], drows.at[slot],
                                  gsem.at[slot]).wait()

        def fire_writeout(g, slot):
            row0 = (ch0 + g * _GK) * _CH
            pltpu.async_copy(srows.at[slot], s_out.at[pl.ds(row0, _GR)],
                             wsem.at[slot])
            pltpu.async_copy(drows.at[slot], d_out.at[pl.ds(row0, _GR)],
                             wsem.at[slot])

        def wait_writeout(g, slot):
            row0 = (ch0 + g * _GK) * _CH
            pltpu.make_async_copy(srows.at[slot], s_out.at[pl.ds(row0, _GR)],
                                  wsem.at[slot]).wait()
            pltpu.make_async_copy(drows.at[slot], d_out.at[pl.ds(row0, _GR)],
                                  wsem.at[slot]).wait()

        for r in range(_RS):
            stage_idx(r, r)
        wait_idx(0, 0)
        fire_gathers(0)

        def body(g, _):
            slot = lax.rem(g, _RS)

            @pl.when(g >= 1)
            def _():
                wait_writeout(g - 1, lax.rem(g - 1, _RS))

            wait_gathers(g, slot)
            fire_writeout(g, slot)

            @pl.when(g + _RS < ng)
            def _():
                stage_idx(g + _RS, slot)

            @pl.when(g + 1 < ng)
            def _():
                nslot = lax.rem(g + 1, _RS)
                wait_idx(g + 1, nslot)
                fire_gathers(nslot)
            return 0

        lax.fori_loop(0, ng, body, 0)
        wait_writeout(ng - 1, lax.rem(ng - 1, _RS))

    return k(pn, src2, dst2)


def _scatter_sc(pe, dst1, zeros):
    mesh = plsc.VectorSubcoreMesh(core_axis_name="c", subcore_axis_name="s")

    @functools.partial(
        pl.kernel,
        out_type=jax.ShapeDtypeStruct((_NC, _N_ACC, _F), jnp.float32),
        mesh=mesh,
        scratch_types=[
            pltpu.VMEM((_RS, _GR), jnp.int32),
            pltpu.VMEM((_RS, _GR, _F), jnp.float32),
            pltpu.VMEM_SHARED((_N_ACC, _F), jnp.float32),
            pltpu.SemaphoreType.DMA((_RS,)),
            pltpu.SemaphoreType.DMA((_RS,)),
        ],
        compiler_params=pltpu.CompilerParams(use_tc_tiling_on_sc=False),
    )
    def k(pe_hbm, dst_hbm, z_hbm, out_hbm, didx, rows, acc, ssem, asem):
        c = lax.axis_index("c")
        s = lax.axis_index("s")
        wid = s * _NC + c
        ch0 = wid * _NCHW
        ng = _real_chunks(wid) // _GK
        pltpu.sync_copy(z_hbm.at[pl.ds(s * _NPS, _NPS)],
                        acc.at[pl.ds(s * _NPS, _NPS)])
        plsc.subcore_barrier()

        def stage(g, slot):
            row0 = (ch0 + g * _GK) * _CH
            pltpu.async_copy(dst_hbm.at[pl.ds(row0, _GR)], didx.at[slot],
                             ssem.at[slot])
            pltpu.async_copy(pe_hbm.at[pl.ds(row0, _GR)], rows.at[slot],
                             ssem.at[slot])

        def wait_stage(g, slot):
            row0 = (ch0 + g * _GK) * _CH
            pltpu.make_async_copy(dst_hbm.at[pl.ds(row0, _GR)],
                                  didx.at[slot], ssem.at[slot]).wait()
            pltpu.make_async_copy(pe_hbm.at[pl.ds(row0, _GR)],
                                  rows.at[slot], ssem.at[slot]).wait()

        def fire_adds(slot):
            for j in range(_GK):
                pltpu.async_copy(rows.at[slot, pl.ds(j * _CH, _CH)],
                                 acc.at[didx.at[slot, pl.ds(j * _CH, _CH)]],
                                 asem.at[slot], add=True)

        def wait_adds(g, slot):
            row0 = (ch0 + g * _GK) * _CH
            pltpu.make_async_copy(pe_hbm.at[pl.ds(row0, _GR)],
                                  rows.at[slot], asem.at[slot]).wait()

        for r in range(_RS - 1):
            stage(r, r)

        def body(g, _):
            slot = lax.rem(g, _RS)
            wait_stage(g, slot)

            @pl.when(g >= 1)
            def _():
                wait_adds(g - 1, lax.rem(g - 1, _RS))

            fire_adds(slot)

            @pl.when(g + _RS - 1 < ng)
            def _():
                stage(g + _RS - 1, lax.rem(g + _RS - 1, _RS))
            return 0

        lax.fori_loop(0, ng, body, 0)
        wait_adds(ng - 1, lax.rem(ng - 1, _RS))
        plsc.subcore_barrier()
        pltpu.sync_copy(acc.at[pl.ds(s * _NPS, _NPS)],
                        out_hbm.at[c].at[pl.ds(s * _NPS, _NPS)])

    return k(pe, dst1, zeros)


def _mlp_tc(p, xs, acc3=None, res=None, br=_BRE, w1_row_splits=None):
    R = xs[0].shape[0]
    grid = R // br
    assert grid * br == R
    w1, b1 = p['inp']
    w2, b2 = p['hidden'][0]
    w3, b3 = p['out']
    fo = w3.shape[1]
    ln = p.get('ln')

    splits = []
    off = 0
    for x in xs:
        splits.append(w1[off:off + x.shape[1]])
        off += x.shape[1]
    w1_acc = w1[off:] if acc3 is not None else None

    nx = len(xs)
    has_acc = acc3 is not None
    has_res = res is not None
    has_ln = ln is not None

    def body(*refs):
        i = 0
        xr = refs[i:i + nx]; i += nx
        if has_acc:
            a0 = refs[i]; a1 = refs[i + 1]; i += 2
        w1r = refs[i:i + nx]; i += nx
        if has_acc:
            w1ar = refs[i]; i += 1
        b1r, w2r, b2r, w3r, b3r = refs[i:i + 5]; i += 5
        if has_ln:
            gr, br_ = refs[i], refs[i + 1]; i += 2
        if has_res:
            rr = refs[i]; i += 1
        out = refs[i]

        f = b1r[...]
        for x, w in zip(xr, w1r):
            f = f + jnp.dot(x[...], w[...], preferred_element_type=jnp.float32)
        if has_acc:
            asum = a0[0] + a1[0]
            f = f + jnp.dot(asum, w1ar[...], preferred_element_type=jnp.float32)
        f = _leaky(f)
        f = _leaky(jnp.dot(f, w2r[...], preferred_element_type=jnp.float32) + b2r[...])
        f = jnp.dot(f, w3r[...], preferred_element_type=jnp.float32) + b3r[...]
        if has_ln:
            mu = jnp.mean(f, axis=1, keepdims=True)
            var = jnp.mean((f - mu) ** 2, axis=1, keepdims=True)
            f = (f - mu) * lax.rsqrt(var + 1e-5) * gr[...] + br_[...]
        if has_res:
            f = f + rr[...]
        out[...] = f

    full = lambda a: pl.BlockSpec(a.shape, lambda i: tuple(0 for _ in a.shape))
    rowblk = lambda a: pl.BlockSpec((br, a.shape[1]), lambda i: (i, 0))

    operands = []
    in_specs = []
    for x in xs:
        operands.append(x); in_specs.append(rowblk(x))
    if has_acc:
        operands += [acc3, acc3]
        in_specs += [pl.BlockSpec((1, br, _F), lambda i: (0, i, 0)),
                     pl.BlockSpec((1, br, _F), lambda i: (1, i, 0))]
    for w in splits:
        operands.append(w); in_specs.append(full(w))
    if has_acc:
        operands.append(w1_acc); in_specs.append(full(w1_acc))
    for a in (b1.reshape(1, -1), w2, b2.reshape(1, -1), w3, b3.reshape(1, -1)):
        operands.append(a); in_specs.append(full(a))
    if has_ln:
        for a in (ln[0].reshape(1, -1), ln[1].reshape(1, -1)):
            operands.append(a); in_specs.append(full(a))
    if has_res:
        operands.append(res); in_specs.append(rowblk(res))

    return pl.pallas_call(
        body,
        grid=(grid,),
        in_specs=in_specs,
        out_specs=pl.BlockSpec((br, fo), lambda i: (i, 0)),
        out_shape=jax.ShapeDtypeStruct((R, fo), jnp.float32),
        compiler_params=pltpu.CompilerParams(
            dimension_semantics=("arbitrary",)),
    )(*operands)


def kernel(nfeatures, efeatures, edge_index, params):
    pad = _E_PAD - _E
    src2 = jnp.pad(edge_index[0], (0, pad)).reshape(_NW * _NCHW, _CH)
    dst1 = jnp.pad(edge_index[1], (0, pad))
    dst2 = dst1.reshape(_NW * _NCHW, _CH)
    zeros = jnp.zeros((_N_ACC, _F), jnp.float32)

    pn = _mlp_tc(params['enc_n'], [nfeatures], br=_BRN)
    pe = _mlp_tc(params['enc_e'], [efeatures], br=_BRE)
    for i in range(2):
        s, d = _gather_sc(pn, src2, dst2)
        pe = _mlp_tc(params['proc_e'][i], [pe, s, d], res=pe, br=_BRE)
        parts = _scatter_sc(pe, dst1, zeros)
        pn = _mlp_tc(params['proc_n'][i], [pn], acc3=parts, res=pn, br=_BRN)
    return _mlp_tc(params['out'], [pn], br=_BRN)


# speedup vs baseline: 6.6774x; 2.2224x over previous
import functools

import jax
import jax.numpy as jnp
from jax import lax
from jax.experimental import pallas as pl
from jax.experimental.pallas import tpu as pltpu
from jax.experimental.pallas import tpu_sc as plsc

_N = 100000
_E = 1600000
_F = 16

_NC = 2
_NS = 16
_NW = _NC * _NS
_CH = 128
_NCHW = 392
_EPW = _NCHW * _CH
_E_PAD = _EPW * _NW
_TOT_CH = _E // _CH
_NP = 102400
_N_ACC = _NP
_NPS = _N_ACC // _NS

_BRE = 2000
_BRN = 1280


def _leaky(x):
    return jnp.where(x >= 0, x, 0.01 * x)


def _real_chunks(wid):
    return jnp.clip(_TOT_CH - wid * _NCHW, 0, _NCHW)


_GK = 4
_GR = _GK * _CH
_RS = 3


def _gather_sc(pn, src2, dst2):
    mesh = plsc.VectorSubcoreMesh(core_axis_name="c", subcore_axis_name="s")

    @functools.partial(
        pl.kernel,
        out_type=(jax.ShapeDtypeStruct((_E, _F), jnp.float32),
                  jax.ShapeDtypeStruct((_E, _F), jnp.float32)),
        mesh=mesh,
        scratch_types=[
            pltpu.VMEM((_RS, _GK, _CH), jnp.int32),
            pltpu.VMEM((_RS, _GK, _CH), jnp.int32),
            pltpu.VMEM((_RS, _GR, _F), jnp.float32),
            pltpu.VMEM((_RS, _GR, _F), jnp.float32),
            pltpu.SemaphoreType.DMA((_RS,)),
            pltpu.SemaphoreType.DMA((_RS,)),
            pltpu.SemaphoreType.DMA((_RS,)),
        ],
        compiler_params=pltpu.CompilerParams(use_tc_tiling_on_sc=False),
    )
    def k(pn_hbm, src_hbm, dst_hbm, s_out, d_out, sidx, didx, srows, drows,
          isem, gsem, wsem):
        wid = lax.axis_index("s") * _NC + lax.axis_index("c")
        ch0 = wid * _NCHW
        ng = _real_chunks(wid) // _GK

        def stage_idx(g, slot):
            pltpu.async_copy(src_hbm.at[pl.ds(ch0 + g * _GK, _GK)],
                             sidx.at[slot], isem.at[slot])
            pltpu.async_copy(dst_hbm.at[pl.ds(ch0 + g * _GK, _GK)],
                             didx.at[slot], isem.at[slot])

        def wait_idx(g, slot):
            pltpu.make_async_copy(src_hbm.at[pl.ds(ch0 + g * _GK, _GK)],
                                  sidx.at[slot], isem.at[slot]).wait()
            pltpu.make_async_copy(dst_hbm.at[pl.ds(ch0 + g * _GK, _GK)],
                                  didx.at[slot], isem.at[slot]).wait()

        def fire_gathers(slot):
            for j in range(_GK):
                pltpu.async_copy(pn_hbm.at[sidx.at[slot, j]],
                                 srows.at[slot, pl.ds(j * _CH, _CH)],
                                 gsem.at[slot])
                pltpu.async_copy(pn_hbm.at[didx.at[slot, j]],
                                 drows.at[slot, pl.ds(j * _CH, _CH)],
                                 gsem.at[slot])

        def wait_gathers(g, slot):
            row0 = (ch0 + g * _GK) * _CH
            pltpu.make_async_copy(s_out.at[pl.ds(row0, _GR)], srows.at[slot],
                                  gsem.at[slot]).wait()
            pltpu.make_async_copy(s_out.at[pl.ds(row0, _GR)], drows.at[slot],
                                  gsem.at[slot]).wait()

        def fire_writeout(g, slot):
            row0 = (ch0 + g * _GK) * _CH
            pltpu.async_copy(srows.at[slot], s_out.at[pl.ds(row0, _GR)],
                             wsem.at[slot])
            pltpu.async_copy(drows.at[slot], d_out.at[pl.ds(row0, _GR)],
                             wsem.at[slot])

        def wait_writeout(g, slot):
            row0 = (ch0 + g * _GK) * _CH
            pltpu.make_async_copy(srows.at[slot], s_out.at[pl.ds(row0, _GR)],
                                  wsem.at[slot]).wait()
            pltpu.make_async_copy(drows.at[slot], d_out.at[pl.ds(row0, _GR)],
                                  wsem.at[slot]).wait()

        for r in range(_RS):
            stage_idx(r, r)
        wait_idx(0, 0)
        fire_gathers(0)

        def body(g, _):
            slot = lax.rem(g, _RS)

            @pl.when(g >= 1)
            def _():
                wait_writeout(g - 1, lax.rem(g - 1, _RS))

            wait_gathers(g, slot)
            fire_writeout(g, slot)

            @pl.when(g + _RS < ng)
            def _():
                stage_idx(g + _RS, slot)

            @pl.when(g + 1 < ng)
            def _():
                nslot = lax.rem(g + 1, _RS)
                wait_idx(g + 1, nslot)
                fire_gathers(nslot)
            return 0

        lax.fori_loop(0, ng, body, 0)
        wait_writeout(ng - 1, lax.rem(ng - 1, _RS))

    return k(pn, src2, dst2)


def _scatter_sc(pe, dst1, zeros):
    mesh = plsc.VectorSubcoreMesh(core_axis_name="c", subcore_axis_name="s")

    @functools.partial(
        pl.kernel,
        out_type=jax.ShapeDtypeStruct((_NC, _N_ACC, _F), jnp.float32),
        mesh=mesh,
        scratch_types=[
            pltpu.VMEM((_RS, _GR), jnp.int32),
            pltpu.VMEM((_RS, _GR, _F), jnp.float32),
            pltpu.VMEM_SHARED((_N_ACC, _F), jnp.float32),
            pltpu.SemaphoreType.DMA((_RS,)),
            pltpu.SemaphoreType.DMA((_RS,)),
        ],
        compiler_params=pltpu.CompilerParams(use_tc_tiling_on_sc=False),
    )
    def k(pe_hbm, dst_hbm, z_hbm, out_hbm, didx, rows, acc, ssem, asem):
        c = lax.axis_index("c")
        s = lax.axis_index("s")
        wid = s * _NC + c
        ch0 = wid * _NCHW
        ng = _real_chunks(wid) // _GK
        pltpu.sync_copy(z_hbm.at[pl.ds(s * _NPS, _NPS)],
                        acc.at[pl.ds(s * _NPS, _NPS)])
        plsc.subcore_barrier()

        def stage(g, slot):
            row0 = (ch0 + g * _GK) * _CH
            pltpu.async_copy(dst_hbm.at[pl.ds(row0, _GR)], didx.at[slot],
                             ssem.at[slot])
            pltpu.async_copy(pe_hbm.at[pl.ds(row0, _GR)], rows.at[slot],
                             ssem.at[slot])

        def wait_stage(g, slot):
            row0 = (ch0 + g * _GK) * _CH
            pltpu.make_async_copy(dst_hbm.at[pl.ds(row0, _GR)],
                                  didx.at[slot], ssem.at[slot]).wait()
            pltpu.make_async_copy(pe_hbm.at[pl.ds(row0, _GR)],
                                  rows.at[slot], ssem.at[slot]).wait()

        def fire_adds(slot):
            for j in range(_GK):
                pltpu.async_copy(rows.at[slot, pl.ds(j * _CH, _CH)],
                                 acc.at[didx.at[slot, pl.ds(j * _CH, _CH)]],
                                 asem.at[slot], add=True)

        def wait_adds(g, slot):
            row0 = (ch0 + g * _GK) * _CH
            pltpu.make_async_copy(pe_hbm.at[pl.ds(row0, _GR)],
                                  rows.at[slot], asem.at[slot]).wait()

        for r in range(_RS - 1):
            stage(r, r)

        def body(g, _):
            slot = lax.rem(g, _RS)
            wait_stage(g, slot)

            @pl.when(g >= 1)
            def _():
                wait_adds(g - 1, lax.rem(g - 1, _RS))

            fire_adds(slot)

            @pl.when(g + _RS - 1 < ng)
            def _():
                stage(g + _RS - 1, lax.rem(g + _RS - 1, _RS))
            return 0

        lax.fori_loop(0, ng, body, 0)
        wait_adds(ng - 1, lax.rem(ng - 1, _RS))
        plsc.subcore_barrier()
        pltpu.sync_copy(acc.at[pl.ds(s * _NPS, _NPS)],
                        out_hbm.at[c].at[pl.ds(s * _NPS, _NPS)])

    return k(pe, dst1, zeros)


_PK = 8


def _kr(w):
    return jnp.kron(jnp.eye(_PK, dtype=w.dtype), w)


def _t8(v):
    return jnp.tile(v, _PK).reshape(1, -1)


def _mlp_tc(p, xs, acc3=None, res=None, br=1000):
    R = xs[0].shape[0]
    grid = R // br
    assert grid * br == R
    w1, b1 = p['inp']
    w2, b2 = p['hidden'][0]
    w3, b3 = p['out']
    ln = p.get('ln')

    splits = []
    off = 0
    for x in xs:
        fi = x.shape[1] // _PK
        splits.append(_kr(w1[off:off + fi]))
        off += fi
    w1_acc = _kr(w1[off:]) if acc3 is not None else None
    kw2, kw3 = _kr(w2), _kr(w3)
    b1p, b2p, b3p = _t8(b1), _t8(b2), _t8(b3)
    fo = kw3.shape[1]

    nx = len(xs)
    has_acc = acc3 is not None
    has_res = res is not None
    has_ln = ln is not None
    if has_ln:
        f_ln = w3.shape[1]
        pmat = jnp.kron(jnp.eye(_PK, dtype=jnp.float32),
                        jnp.full((f_ln, f_ln), 1.0 / f_ln, jnp.float32))
        gp, betap = _t8(ln[0]), _t8(ln[1])

    def body(*refs):
        i = 0
        xr = refs[i:i + nx]; i += nx
        if has_acc:
            a0 = refs[i]; a1 = refs[i + 1]; i += 2
        w1r = refs[i:i + nx]; i += nx
        if has_acc:
            w1ar = refs[i]; i += 1
        b1r, w2r, b2r, w3r, b3r = refs[i:i + 5]; i += 5
        if has_ln:
            pr, gr, br_ = refs[i], refs[i + 1], refs[i + 2]; i += 3
        if has_res:
            rr = refs[i]; i += 1
        out = refs[i]

        f = b1r[...]
        for x, w in zip(xr, w1r):
            f = f + jnp.dot(x[...], w[...], preferred_element_type=jnp.float32)
        if has_acc:
            asum = a0[0] + a1[0]
            f = f + jnp.dot(asum, w1ar[...], preferred_element_type=jnp.float32)
        f = _leaky(f)
        f = _leaky(jnp.dot(f, w2r[...], preferred_element_type=jnp.float32) + b2r[...])
        f = jnp.dot(f, w3r[...], preferred_element_type=jnp.float32) + b3r[...]
        if has_ln:
            mu = jnp.dot(f, pr[...], preferred_element_type=jnp.float32)
            m2 = jnp.dot(f * f, pr[...], preferred_element_type=jnp.float32)
            var = m2 - mu * mu
            f = (f - mu) * lax.rsqrt(var + 1e-5) * gr[...] + br_[...]
        if has_res:
            f = f + rr[...]
        out[...] = f

    full = lambda a: pl.BlockSpec(a.shape, lambda i: tuple(0 for _ in a.shape))
    rowblk = lambda a: pl.BlockSpec((br, a.shape[1]), lambda i: (i, 0))

    operands = []
    in_specs = []
    for x in xs:
        operands.append(x); in_specs.append(rowblk(x))
    if has_acc:
        operands += [acc3, acc3]
        in_specs += [pl.BlockSpec((1, br, _PK * _F), lambda i: (0, i, 0)),
                     pl.BlockSpec((1, br, _PK * _F), lambda i: (1, i, 0))]
    for w in splits:
        operands.append(w); in_specs.append(full(w))
    if has_acc:
        operands.append(w1_acc); in_specs.append(full(w1_acc))
    for a in (b1p, kw2, b2p, kw3, b3p):
        operands.append(a); in_specs.append(full(a))
    if has_ln:
        for a in (pmat, gp, betap):
            operands.append(a); in_specs.append(full(a))
    if has_res:
        operands.append(res); in_specs.append(rowblk(res))

    return pl.pallas_call(
        body,
        grid=(grid,),
        in_specs=in_specs,
        out_specs=pl.BlockSpec((br, fo), lambda i: (i, 0)),
        out_shape=jax.ShapeDtypeStruct((R, fo), jnp.float32),
        compiler_params=pltpu.CompilerParams(
            dimension_semantics=("arbitrary",)),
    )(*operands)


def kernel(nfeatures, efeatures, edge_index, params):
    pad = _E_PAD - _E
    src2 = jnp.pad(edge_index[0], (0, pad)).reshape(_NW * _NCHW, _CH)
    dst1 = jnp.pad(edge_index[1], (0, pad))
    zeros = jnp.zeros((_N_ACC, _F), jnp.float32)

    dst2 = dst1.reshape(_NW * _NCHW, _CH)
    ep = _E // _PK
    np_ = _NP // _PK
    nf = jnp.pad(nfeatures, ((0, _NP - _N), (0, 0))).reshape(np_, _PK * 8)
    ef = efeatures.reshape(ep, _PK * 4)

    pn = _mlp_tc(params['enc_n'], [nf], br=_BRN)
    pe = _mlp_tc(params['enc_e'], [ef], br=_BRE)
    for i in range(2):
        s, d = _gather_sc(pn.reshape(_NP, _F), src2, dst2)
        sp = s.reshape(ep, _PK * _F)
        dp = d.reshape(ep, _PK * _F)
        pe = _mlp_tc(params['proc_e'][i], [pe, sp, dp], res=pe, br=_BRE)
        parts = _scatter_sc(pe.reshape(_E, _F), dst1, zeros)
        pp = parts.reshape(_NC, _N_ACC // _PK, _PK * _F)
        pn = _mlp_tc(params['proc_n'][i], [pn], acc3=pp, res=pn, br=_BRN)
    return _mlp_tc(params['out'], [pn], br=_BRN).reshape(_NP, 2)[:_N]


# speedup vs baseline: 10.6443x; 1.5941x over previous
import functools

import jax
import jax.numpy as jnp
from jax import lax
from jax.experimental import pallas as pl
from jax.experimental.pallas import tpu as pltpu
from jax.experimental.pallas import tpu_sc as plsc

_N = 100000
_E = 1600000
_F = 16

_NC = 2
_NS = 16
_NW = _NC * _NS
_CH = 128
_NCHW = 392
_EPW = _NCHW * _CH
_E_PAD = _EPW * _NW
_TOT_CH = _E // _CH
_NP = 102400
_N_ACC = _NP
_NPS = _N_ACC // _NS

_BRE = 2000
_BRN = 1280


def _leaky(x):
    return jnp.where(x >= 0, x, 0.01 * x)


def _real_chunks(wid):
    return jnp.clip(_TOT_CH - wid * _NCHW, 0, _NCHW)


_GK = 4
_GR = _GK * _CH
_RS = 3


_CE = 2000


def _fmt_ef_sc(efT):
    mesh = plsc.VectorSubcoreMesh(core_axis_name="c", subcore_axis_name="s")
    epw = _E // _NW
    nch = epw // _CE

    @functools.partial(
        pl.kernel,
        out_type=jax.ShapeDtypeStruct((_E * 4,), jnp.float32),
        mesh=mesh,
        scratch_types=[
            pltpu.VMEM((2, 4, _CE), jnp.float32),
            pltpu.VMEM((2, 4 * _CE), jnp.float32),
            pltpu.SemaphoreType.DMA((2,)),
            pltpu.SemaphoreType.DMA((2,)),
        ],
        compiler_params=pltpu.CompilerParams(use_tc_tiling_on_sc=False,
                                             needs_layout_passes=False),
    )
    def k(ef_hbm, out_hbm, col, rowb, ssem, wsem):
        wid = lax.axis_index("s") * _NC + lax.axis_index("c")
        e0w = wid * epw
        iota4 = lax.iota(jnp.int32, 16) * 4

        def stage(c, slot):
            for f in range(4):
                pltpu.async_copy(ef_hbm.at[f, pl.ds(e0w + c * _CE, _CE)],
                                 col.at[slot, f], ssem.at[slot])

        def wait_stage(c, slot):
            for f in range(4):
                pltpu.make_async_copy(ef_hbm.at[f, pl.ds(e0w + c * _CE, _CE)],
                                      col.at[slot, f], ssem.at[slot]).wait()

        def wait_write(c, slot):
            pltpu.make_async_copy(rowb.at[slot],
                                  out_hbm.at[pl.ds((e0w + c * _CE) * 4,
                                                   4 * _CE)],
                                  wsem.at[slot]).wait()

        stage(0, 0)

        def body(c, _):
            slot = lax.rem(c, 2)

            @pl.when(c + 1 < nch)
            def _():
                stage(c + 1, 1 - slot)

            wait_stage(c, slot)

            @pl.when(c >= 2)
            def _():
                wait_write(c - 2, slot)

            for f in range(4):
                for blk in range(_CE // 16):
                    vals = col[slot, f, pl.ds(blk * 16, 16)]
                    plsc.store_scatter(rowb.at[slot],
                                       [iota4 + (blk * 64 + f)], vals)
            pltpu.async_copy(rowb.at[slot],
                             out_hbm.at[pl.ds((e0w + c * _CE) * 4, 4 * _CE)],
                             wsem.at[slot])
            return 0

        lax.fori_loop(0, nch, body, 0)
        wait_write(nch - 2, lax.rem(nch - 2, 2))
        wait_write(nch - 1, lax.rem(nch - 1, 2))

    return k(efT)


def _gather_sc(pn, src2, dst2):
    mesh = plsc.VectorSubcoreMesh(core_axis_name="c", subcore_axis_name="s")

    @functools.partial(
        pl.kernel,
        out_type=(jax.ShapeDtypeStruct((_E, _F), jnp.float32),
                  jax.ShapeDtypeStruct((_E, _F), jnp.float32)),
        mesh=mesh,
        scratch_types=[
            pltpu.VMEM((_RS, _GK, _CH), jnp.int32),
            pltpu.VMEM((_RS, _GK, _CH), jnp.int32),
            pltpu.VMEM((_RS, _GR, _F), jnp.float32),
            pltpu.VMEM((_RS, _GR, _F), jnp.float32),
            pltpu.SemaphoreType.DMA((_RS,)),
            pltpu.SemaphoreType.DMA((_RS,)),
            pltpu.SemaphoreType.DMA((_RS,)),
        ],
        compiler_params=pltpu.CompilerParams(use_tc_tiling_on_sc=False),
    )
    def k(pn_hbm, src_hbm, dst_hbm, s_out, d_out, sidx, didx, srows, drows,
          isem, gsem, wsem):
        wid = lax.axis_index("s") * _NC + lax.axis_index("c")
        ch0 = wid * _NCHW
        ng = _real_chunks(wid) // _GK

        def stage_idx(g, slot):
            pltpu.async_copy(src_hbm.at[pl.ds(ch0 + g * _GK, _GK)],
                             sidx.at[slot], isem.at[slot])
            pltpu.async_copy(dst_hbm.at[pl.ds(ch0 + g * _GK, _GK)],
                             didx.at[slot], isem.at[slot])

        def wait_idx(g, slot):
            pltpu.make_async_copy(src_hbm.at[pl.ds(ch0 + g * _GK, _GK)],
                                  sidx.at[slot], isem.at[slot]).wait()
            pltpu.make_async_copy(dst_hbm.at[pl.ds(ch0 + g * _GK, _GK)],
                                  didx.at[slot], isem.at[slot]).wait()

        def fire_gathers(slot):
            for j in range(_GK):
                pltpu.async_copy(pn_hbm.at[sidx.at[slot, j]],
                                 srows.at[slot, pl.ds(j * _CH, _CH)],
                                 gsem.at[slot])
                pltpu.async_copy(pn_hbm.at[didx.at[slot, j]],
                                 drows.at[slot, pl.ds(j * _CH, _CH)],
                                 gsem.at[slot])

        def wait_gathers(g, slot):
            row0 = (ch0 + g * _GK) * _CH
            pltpu.make_async_copy(s_out.at[pl.ds(row0, _GR)], srows.at[slot],
                                  gsem.at[slot]).wait()
            pltpu.make_async_copy(s_out.at[pl.ds(row0, _GR)], drows.at[slot],
                                  gsem.at[slot]).wait()

        def fire_writeout(g, slot):
            row0 = (ch0 + g * _GK) * _CH
            pltpu.async_copy(srows.at[slot], s_out.at[pl.ds(row0, _GR)],
                             wsem.at[slot])
            pltpu.async_copy(drows.at[slot], d_out.at[pl.ds(row0, _GR)],
                             wsem.at[slot])

        def wait_writeout(g, slot):
            row0 = (ch0 + g * _GK) * _CH
            pltpu.make_async_copy(srows.at[slot], s_out.at[pl.ds(row0, _GR)],
                                  wsem.at[slot]).wait()
            pltpu.make_async_copy(drows.at[slot], d_out.at[pl.ds(row0, _GR)],
                                  wsem.at[slot]).wait()

        for r in range(_RS):
            stage_idx(r, r)
        wait_idx(0, 0)
        fire_gathers(0)

        def body(g, _):
            slot = lax.rem(g, _RS)

            @pl.when(g >= 1)
            def _():
                wait_writeout(g - 1, lax.rem(g - 1, _RS))

            wait_gathers(g, slot)
            fire_writeout(g, slot)

            @pl.when(g + _RS < ng)
            def _():
                stage_idx(g + _RS, slot)

            @pl.when(g + 1 < ng)
            def _():
                nslot = lax.rem(g + 1, _RS)
                wait_idx(g + 1, nslot)
                fire_gathers(nslot)
            return 0

        lax.fori_loop(0, ng, body, 0)
        wait_writeout(ng - 1, lax.rem(ng - 1, _RS))

    return k(pn, src2, dst2)


def _scatter_sc(pe, dst1, zeros):
    mesh = plsc.VectorSubcoreMesh(core_axis_name="c", subcore_axis_name="s")

    @functools.partial(
        pl.kernel,
        out_type=jax.ShapeDtypeStruct((_NC, _N_ACC, _F), jnp.float32),
        mesh=mesh,
        scratch_types=[
            pltpu.VMEM((_RS, _GR), jnp.int32),
            pltpu.VMEM((_RS, _GR, _F), jnp.float32),
            pltpu.VMEM_SHARED((_N_ACC, _F), jnp.float32),
            pltpu.SemaphoreType.DMA((_RS,)),
            pltpu.SemaphoreType.DMA((_RS,)),
        ],
        compiler_params=pltpu.CompilerParams(use_tc_tiling_on_sc=False),
    )
    def k(pe_hbm, dst_hbm, z_hbm, out_hbm, didx, rows, acc, ssem, asem):
        c = lax.axis_index("c")
        s = lax.axis_index("s")
        wid = s * _NC + c
        ch0 = wid * _NCHW
        ng = _real_chunks(wid) // _GK
        pltpu.sync_copy(z_hbm.at[pl.ds(s * _NPS, _NPS)],
                        acc.at[pl.ds(s * _NPS, _NPS)])
        plsc.subcore_barrier()

        def stage(g, slot):
            row0 = (ch0 + g * _GK) * _CH
            pltpu.async_copy(dst_hbm.at[pl.ds(row0, _GR)], didx.at[slot],
                             ssem.at[slot])
            pltpu.async_copy(pe_hbm.at[pl.ds(row0, _GR)], rows.at[slot],
                             ssem.at[slot])

        def wait_stage(g, slot):
            row0 = (ch0 + g * _GK) * _CH
            pltpu.make_async_copy(dst_hbm.at[pl.ds(row0, _GR)],
                                  didx.at[slot], ssem.at[slot]).wait()
            pltpu.make_async_copy(pe_hbm.at[pl.ds(row0, _GR)],
                                  rows.at[slot], ssem.at[slot]).wait()

        def fire_adds(slot):
            for j in range(_GK):
                pltpu.async_copy(rows.at[slot, pl.ds(j * _CH, _CH)],
                                 acc.at[didx.at[slot, pl.ds(j * _CH, _CH)]],
                                 asem.at[slot], add=True)

        def wait_adds(g, slot):
            row0 = (ch0 + g * _GK) * _CH
            pltpu.make_async_copy(pe_hbm.at[pl.ds(row0, _GR)],
                                  rows.at[slot], asem.at[slot]).wait()

        for r in range(_RS - 1):
            stage(r, r)

        def body(g, _):
            slot = lax.rem(g, _RS)
            wait_stage(g, slot)

            @pl.when(g >= 1)
            def _():
                wait_adds(g - 1, lax.rem(g - 1, _RS))

            fire_adds(slot)

            @pl.when(g + _RS - 1 < ng)
            def _():
                stage(g + _RS - 1, lax.rem(g + _RS - 1, _RS))
            return 0

        lax.fori_loop(0, ng, body, 0)
        wait_adds(ng - 1, lax.rem(ng - 1, _RS))
        plsc.subcore_barrier()
        pltpu.sync_copy(acc.at[pl.ds(s * _NPS, _NPS)],
                        out_hbm.at[c].at[pl.ds(s * _NPS, _NPS)])

    return k(pe, dst1, zeros)


_PK = 8


def _kr(w, pk=_PK):
    return jnp.kron(jnp.eye(pk, dtype=w.dtype), w)


def _t8(v, pk=_PK):
    return jnp.tile(v, pk).reshape(1, -1)


def _mlp_tc(p, xs, acc3=None, res=None, br=1000, pk=_PK, tr3d=False):
    R = xs[0].shape[0]
    grid = R // br
    assert grid * br == R
    w1, b1 = p['inp']
    w2, b2 = p['hidden'][0]
    w3, b3 = p['out']
    ln = p.get('ln')

    splits = []
    off = 0
    for x in xs:
        fi = x.shape[2] if tr3d else x.shape[1] // pk
        splits.append(_kr(w1[off:off + fi], pk))
        off += fi
    w1_acc = _kr(w1[off:], pk) if acc3 is not None else None
    kw2, kw3 = _kr(w2, pk), _kr(w3, pk)
    b1p, b2p, b3p = _t8(b1, pk), _t8(b2, pk), _t8(b3, pk)
    fo = kw3.shape[1]
    rep = pk // _PK

    nx = len(xs)
    has_acc = acc3 is not None
    has_res = res is not None
    has_ln = ln is not None
    if has_ln:
        f_ln = w3.shape[1]
        pmat = jnp.kron(jnp.eye(pk, dtype=jnp.float32),
                        jnp.full((f_ln, f_ln), 1.0 / f_ln, jnp.float32))
        gp, betap = _t8(ln[0], pk), _t8(ln[1], pk)

    def body(*refs):
        i = 0
        xr = refs[i:i + nx]; i += nx
        if has_acc:
            a0 = refs[i]; a1 = refs[i + 1]; i += 2
        w1r = refs[i:i + nx]; i += nx
        if has_acc:
            w1ar = refs[i]; i += 1
        b1r, w2r, b2r, w3r, b3r = refs[i:i + 5]; i += 5
        if has_ln:
            pr, gr, br_ = refs[i], refs[i + 1], refs[i + 2]; i += 3
        if has_res:
            rr = refs[i]; i += 1
        out = refs[i]

        f = b1r[...]
        for x, w in zip(xr, w1r):
            xv = x[...]
            if tr3d:
                xv = xv.transpose(0, 2, 1).reshape(-1, pk * xv.shape[1])
            f = f + jnp.dot(xv, w[...], preferred_element_type=jnp.float32)
        if has_acc:
            asum = a0[0] + a1[0]
            f = f + jnp.dot(asum, w1ar[...], preferred_element_type=jnp.float32)
        f = _leaky(f)
        f = _leaky(jnp.dot(f, w2r[...], preferred_element_type=jnp.float32) + b2r[...])
        f = jnp.dot(f, w3r[...], preferred_element_type=jnp.float32) + b3r[...]
        if has_ln:
            mu = jnp.dot(f, pr[...], preferred_element_type=jnp.float32)
            m2 = jnp.dot(f * f, pr[...], preferred_element_type=jnp.float32)
            var = m2 - mu * mu
            f = (f - mu) * lax.rsqrt(var + 1e-5) * gr[...] + br_[...]
        if has_res:
            f = f + rr[...]
        if rep > 1:
            f = f.reshape(f.shape[0] * rep, fo // rep)
        out[...] = f

    full = lambda a: pl.BlockSpec(a.shape, lambda i: tuple(0 for _ in a.shape))
    if tr3d:
        rowblk = lambda a: pl.BlockSpec((br, a.shape[1], 128),
                                        lambda i: (i, 0, 0))
    else:
        rowblk = lambda a: pl.BlockSpec((br, a.shape[1]), lambda i: (i, 0))

    operands = []
    in_specs = []
    for x in xs:
        operands.append(x); in_specs.append(rowblk(x))
    if has_acc:
        operands += [acc3, acc3]
        in_specs += [pl.BlockSpec((1, br, _PK * _F), lambda i: (0, i, 0)),
                     pl.BlockSpec((1, br, _PK * _F), lambda i: (1, i, 0))]
    for w in splits:
        operands.append(w); in_specs.append(full(w))
    if has_acc:
        operands.append(w1_acc); in_specs.append(full(w1_acc))
    for a in (b1p, kw2, b2p, kw3, b3p):
        operands.append(a); in_specs.append(full(a))
    if has_ln:
        for a in (pmat, gp, betap):
            operands.append(a); in_specs.append(full(a))
    if has_res:
        operands.append(res); in_specs.append(rowblk(res))

    if tr3d:
        obr, ofo = br * 128 // pk, fo
        orows = R * 128 // pk
    else:
        obr, ofo = br * rep, fo // rep
        orows = R * rep
    return pl.pallas_call(
        body,
        grid=(grid,),
        in_specs=in_specs,
        out_specs=pl.BlockSpec((obr, ofo), lambda i: (i, 0)),
        out_shape=jax.ShapeDtypeStruct((orows, ofo), jnp.float32),
        compiler_params=pltpu.CompilerParams(
            dimension_semantics=("arbitrary",)),
    )(*operands)


def kernel(nfeatures, efeatures, edge_index, params):
    pad = _E_PAD - _E
    src2 = jnp.pad(edge_index[0], (0, pad)).reshape(_NW * _NCHW, _CH)
    dst1 = jnp.pad(edge_index[1], (0, pad))
    zeros = jnp.zeros((_N_ACC, _F), jnp.float32)

    dst2 = dst1.reshape(_NW * _NCHW, _CH)
    ep = _E // _PK
    np_ = _NP // _PK
    nf = nfeatures.reshape(_N // 16, 128)
    ef = _fmt_ef_sc(efeatures.T).reshape(_E // 32, 128)

    pn = _mlp_tc(params['enc_n'], [nf], br=_N // 16, pk=16)
    pn = jnp.pad(pn, ((0, np_ - _N // _PK), (0, 0)))
    pe = _mlp_tc(params['enc_e'], [ef], br=1000, pk=32)
    for i in range(2):
        s, d = _gather_sc(pn.reshape(_NP, _F), src2, dst2)
        sp = s.reshape(ep, _PK * _F)
        dp = d.reshape(ep, _PK * _F)
        pe = _mlp_tc(params['proc_e'][i], [pe, sp, dp], res=pe, br=_BRE)
        parts = _scatter_sc(pe.reshape(_E, _F), dst1, zeros)
        pp = parts.reshape(_NC, _N_ACC // _PK, _PK * _F)
        pn = _mlp_tc(params['proc_n'][i], [pn], acc3=pp, res=pn, br=_BRN)
    return _mlp_tc(params['out'], [pn], br=_BRN).reshape(_NP, 2)[:_N]
